# baseline (speedup 1.0000x reference)
import functools

import jax
import jax.numpy as jnp
from jax import lax
from jax.experimental import pallas as pl
from jax.experimental.pallas import tpu as pltpu
from jax.experimental.pallas import tpu_sc as plsc

_B, _C, _H, _W = 4, 192, 224, 224
_P = 64
_PLANE = _H * 128
_NW = 32
_ROWS = _B * _P
_RPW = _ROWS // _NW
_LANES = 16
_CPR = _C // _LANES
_HALF = _C // 2
_NT = _RPW * _C // 128

_MESH = plsc.VectorSubcoreMesh(core_axis_name="c", subcore_axis_name="s")


def _point_scalars(pts_v, lanes):
    pv = pts_v[...]
    zero = jnp.zeros((16,), jnp.int32)
    out = []
    for j in range(_RPW):
        xj = jnp.sum(jnp.where(lanes == 2 * j, pv, zero))
        yj = jnp.sum(jnp.where(lanes == 2 * j + 1, pv, zero))
        out.append((xj, yj))
    return out


@jax.jit
def _crop_lo(imgs_rows, pts_flat):
    @functools.partial(
        pl.kernel,
        mesh=_MESH,
        out_type=jax.ShapeDtypeStruct((_ROWS * _C,), jnp.float32),
        scratch_types=[
            pltpu.VMEM((_LANES,), jnp.int32),
            pltpu.VMEM((2 * _RPW, _HALF), jnp.int32),
            pltpu.VMEM((2, _C, 128), jnp.float32),
            pltpu.VMEM((_RPW * _C,), jnp.float32),
            pltpu.SemaphoreType.DMA,
            pltpu.SemaphoreType.DMA,
        ],
        compiler_params=pltpu.CompilerParams(needs_layout_passes=False),
    )
    def crop_lo(imgs_hbm, pts_hbm, out_hbm, pts_v, ridx_v, stage_v, out_v,
                sem0, sem1):
        lo = imgs_hbm.at[:, pl.ds(0, 128)]
        sems = (sem0, sem1)
        wid = lax.axis_index("s") * 2 + lax.axis_index("c")
        pltpu.sync_copy(pts_hbm.at[pl.ds(wid * 16, 16)], pts_v)
        base = (wid // 8) * _C
        lanes = lax.iota(jnp.int32, 16)
        xys = _point_scalars(pts_v, lanes)
        xlos = []
        for j, (xj, yj) in enumerate(xys):
            xlos.append(jnp.minimum(xj, 127))
            for h in range(2):
                for k in range(_HALF // 16):
                    c = h * _HALF + k * 16 + lanes
                    ridx_v[2 * j + h, pl.ds(k * 16, 16)] = (
                        (base + c) * _H + yj
                    )

        def fire(j, buf):
            return [
                pltpu.async_copy(
                    lo.at[ridx_v.at[2 * j + h]],
                    stage_v.at[buf, pl.ds(h * _HALF, _HALF)],
                    sems[buf],
                )
                for h in range(2)
            ]

        pending = fire(0, 0)
        for j in range(_RPW):
            nxt = fire(j + 1, (j + 1) % 2) if j + 1 < _RPW else None
            for cpy in pending:
                cpy.wait()
            buf = j % 2
            xvec = jnp.full((16,), xlos[j], jnp.int32)
            for ci in range(_CPR):
                vals = plsc.load_gather(
                    stage_v.at[buf], [ci * 16 + lanes, xvec]
                )
                out_v[pl.ds(j * _C + ci * 16, 16)] = vals
            pending = nxt
        pltpu.sync_copy(out_v, out_hbm.at[pl.ds(wid * _NT * 128, _NT * 128)])

    return crop_lo(imgs_rows, pts_flat)


@jax.jit
def _crop_hi(hi_flat, lo_out, pts_flat):
    @functools.partial(
        pl.kernel,
        mesh=_MESH,
        out_type=jax.ShapeDtypeStruct((_ROWS * _C,), jnp.float32),
        scratch_types=[
            pltpu.VMEM((_LANES,), jnp.int32),
            pltpu.VMEM((_NT, 128), jnp.int32),
            pltpu.VMEM((_NT * 128,), jnp.float32),
            pltpu.VMEM((_NT * 128,), jnp.float32),
            pltpu.SemaphoreType.DMA,
        ],
        compiler_params=pltpu.CompilerParams(needs_layout_passes=False),
    )
    def crop_hi(hi_hbm, lo_hbm, pts_hbm, out_hbm, pts_v, ihi_v, chi_v,
                out_v, sem):
        wid = lax.axis_index("s") * 2 + lax.axis_index("c")
        pltpu.sync_copy(pts_hbm.at[pl.ds(wid * 16, 16)], pts_v)
        base = (wid // 8) * _C
        lanes = lax.iota(jnp.int32, 16)
        xys = _point_scalars(pts_v, lanes)
        sels = []
        for j, (xj, yj) in enumerate(xys):
            sels.append(xj < 128)
            phi = yj * 128 + jnp.maximum(xj - 128, 0)
            for ci in range(_CPR):
                k = j * _CPR + ci
                ihi_v[k // 8, pl.ds((k % 8) * 16, 16)] = (
                    (base + ci * 16 + lanes) * _PLANE + phi
                )
        copies = [
            pltpu.async_copy(
                hi_hbm.at[ihi_v.at[t]], chi_v.at[pl.ds(t * 128, 128)], sem
            )
            for t in range(_NT)
        ]
        pltpu.sync_copy(lo_hbm.at[pl.ds(wid * _NT * 128, _NT * 128)], out_v)
        for cpy in copies:
            cpy.wait()
        for j in range(_RPW):
            sel = jnp.full((16,), sels[j], jnp.bool_)
            for ci in range(_CPR):
                o = j * _C + ci * 16
                out_v[pl.ds(o, 16)] = jnp.where(
                    sel, out_v[pl.ds(o, 16)], chi_v[pl.ds(o, 16)]
                )
        pltpu.sync_copy(out_v, out_hbm.at[pl.ds(wid * _NT * 128, _NT * 128)])

    return crop_hi(hi_flat, lo_out, pts_flat)


_RB = 8192


@jax.jit
def _hi_slab(imgs_rows):
    def body(i_ref, o_ref):
        o_ref[:, : _W - 128] = i_ref[:, 128:_W]

    return pl.pallas_call(
        body,
        grid=(_B * _C * _H // _RB,),
        in_specs=[pl.BlockSpec((_RB, _W), lambda i: (i, 0))],
        out_specs=pl.BlockSpec((_RB, 128), lambda i: (i, 0)),
        out_shape=jax.ShapeDtypeStruct((_B * _C * _H, 128), jnp.float32),
    )(imgs_rows)


def kernel(imgs, batch_points):
    imgs_rows = imgs.reshape(_B * _C * _H, _W)
    pts_flat = batch_points.astype(jnp.int32).reshape(-1)
    hi_flat = _hi_slab(imgs_rows).reshape(-1)
    lo_out = _crop_lo(imgs_rows, pts_flat)
    out = _crop_hi(hi_flat, lo_out, pts_flat)
    return out.reshape(_ROWS, _C), _ROWS


# speedup vs baseline: 1.2946x; 1.2946x over previous
import functools

import jax
import jax.numpy as jnp
from jax import lax
from jax.experimental import pallas as pl
from jax.experimental.pallas import tpu as pltpu
from jax.experimental.pallas import tpu_sc as plsc

_B, _C, _H, _W = 4, 192, 224, 224
_P = 64
_PLANE = _H * 128
_NW = 32
_ROWS = _B * _P
_RPW = _ROWS // _NW
_LANES = 16
_CPR = _C // _LANES
_HALF = _C // 2
_NT = _RPW * _C // 128

_MESH = plsc.VectorSubcoreMesh(core_axis_name="c", subcore_axis_name="s")


def _point_scalars(pts_v, lanes):
    pv = pts_v[...]
    zero = jnp.zeros((16,), jnp.int32)
    out = []
    for j in range(_RPW):
        xj = jnp.sum(jnp.where(lanes == 2 * j, pv, zero))
        yj = jnp.sum(jnp.where(lanes == 2 * j + 1, pv, zero))
        out.append((xj, yj))
    return out


@jax.jit
def _crop_lo(imgs_rows, pts_flat):
    @functools.partial(
        pl.kernel,
        mesh=_MESH,
        out_type=jax.ShapeDtypeStruct((_ROWS * _C,), jnp.float32),
        scratch_types=[
            pltpu.VMEM((_LANES,), jnp.int32),
            pltpu.VMEM((2 * _RPW, _HALF), jnp.int32),
            pltpu.VMEM((2, _C, 128), jnp.float32),
            pltpu.VMEM((_RPW * _C,), jnp.float32),
            pltpu.SemaphoreType.DMA,
            pltpu.SemaphoreType.DMA,
        ],
        compiler_params=pltpu.CompilerParams(needs_layout_passes=False),
    )
    def crop_lo(imgs_hbm, pts_hbm, out_hbm, pts_v, ridx_v, stage_v, out_v,
                sem0, sem1):
        lo = imgs_hbm.at[:, pl.ds(0, 128)]
        sems = (sem0, sem1)
        wid = lax.axis_index("s") * 2 + lax.axis_index("c")
        pltpu.sync_copy(pts_hbm.at[pl.ds(wid * 16, 16)], pts_v)
        base = (wid // 8) * _C
        lanes = lax.iota(jnp.int32, 16)
        xys = _point_scalars(pts_v, lanes)
        xlos = []
        for j, (xj, yj) in enumerate(xys):
            xlos.append(jnp.minimum(xj, 127))
            for h in range(2):
                for k in range(_HALF // 16):
                    c = h * _HALF + k * 16 + lanes
                    ridx_v[2 * j + h, pl.ds(k * 16, 16)] = (
                        (base + c) * _H + yj
                    )

        def fire(j, buf):
            return [
                pltpu.async_copy(
                    lo.at[ridx_v.at[2 * j + h]],
                    stage_v.at[buf, pl.ds(h * _HALF, _HALF)],
                    sems[buf],
                )
                for h in range(2)
            ]

        pending = fire(0, 0)
        for j in range(_RPW):
            nxt = fire(j + 1, (j + 1) % 2) if j + 1 < _RPW else None
            for cpy in pending:
                cpy.wait()
            buf = j % 2
            xvec = jnp.full((16,), xlos[j], jnp.int32)
            for ci in range(_CPR):
                vals = plsc.load_gather(
                    stage_v.at[buf], [ci * 16 + lanes, xvec]
                )
                out_v[pl.ds(j * _C + ci * 16, 16)] = vals
            pending = nxt
        pltpu.sync_copy(out_v, out_hbm.at[pl.ds(wid * _NT * 128, _NT * 128)])

    return crop_lo(imgs_rows, pts_flat)


@jax.jit
def _crop_hi(hi_flat, lo_out, pts_flat):
    @functools.partial(
        pl.kernel,
        mesh=_MESH,
        out_type=jax.ShapeDtypeStruct((_ROWS * _C,), jnp.float32),
        scratch_types=[
            pltpu.VMEM((_LANES,), jnp.int32),
            pltpu.VMEM((_NT, 128), jnp.int32),
            pltpu.VMEM((_NT * 128,), jnp.float32),
            pltpu.VMEM((_NT * 128,), jnp.float32),
            pltpu.SemaphoreType.DMA,
        ],
        compiler_params=pltpu.CompilerParams(needs_layout_passes=False),
    )
    def crop_hi(hi_hbm, lo_hbm, pts_hbm, out_hbm, pts_v, ihi_v, chi_v,
                out_v, sem):
        wid = lax.axis_index("s") * 2 + lax.axis_index("c")
        pltpu.sync_copy(pts_hbm.at[pl.ds(wid * 16, 16)], pts_v)
        base = (wid // 8) * _C
        lanes = lax.iota(jnp.int32, 16)
        xys = _point_scalars(pts_v, lanes)
        sels = []
        for j, (xj, yj) in enumerate(xys):
            sels.append(xj < 128)
            phi = yj * 128 + jnp.maximum(xj - 128, 0)
            for ci in range(_CPR):
                k = j * _CPR + ci
                ihi_v[k // 8, pl.ds((k % 8) * 16, 16)] = (
                    (base + ci * 16 + lanes) * _PLANE + phi
                )
        copies = [
            pltpu.async_copy(
                hi_hbm.at[ihi_v.at[t]], chi_v.at[pl.ds(t * 128, 128)], sem
            )
            for t in range(_NT)
        ]
        pltpu.sync_copy(lo_hbm.at[pl.ds(wid * _NT * 128, _NT * 128)], out_v)
        for cpy in copies:
            cpy.wait()
        for j in range(_RPW):
            sel = jnp.full((16,), sels[j], jnp.bool_)
            for ci in range(_CPR):
                o = j * _C + ci * 16
                out_v[pl.ds(o, 16)] = jnp.where(
                    sel, out_v[pl.ds(o, 16)], chi_v[pl.ds(o, 16)]
                )
        pltpu.sync_copy(out_v, out_hbm.at[pl.ds(wid * _NT * 128, _NT * 128)])

    return crop_hi(hi_flat, lo_out, pts_flat)


_RB = 24576


@jax.jit
def _hi_slab(imgs_rows):
    def body(i_ref, o_ref):
        o_ref[...] = i_ref[...]

    return pl.pallas_call(
        body,
        grid=(_B * _C * _H // _RB,),
        in_specs=[pl.BlockSpec((_RB, 128), lambda i: (i, 1))],
        out_specs=pl.BlockSpec((_RB, 128), lambda i: (i, 0)),
        out_shape=jax.ShapeDtypeStruct((_B * _C * _H, 128), jnp.float32),
    )(imgs_rows)


def kernel(imgs, batch_points):
    imgs_rows = imgs.reshape(_B * _C * _H, _W)
    pts_flat = batch_points.astype(jnp.int32).reshape(-1)
    hi_flat = _hi_slab(imgs_rows).reshape(-1)
    lo_out = _crop_lo(imgs_rows, pts_flat)
    out = _crop_hi(hi_flat, lo_out, pts_flat)
    return out.reshape(_ROWS, _C), _ROWS
